# Initial kernel scaffold; baseline (speedup 1.0000x reference)
import jax
import jax.numpy as jnp
from jax.experimental import pallas as pl


def kernel(frames):
    raise NotImplementedError("write your pallas kernel here")


# speedup vs baseline: 1.6946x; 1.6946x over previous
import jax
import jax.numpy as jnp
import numpy as np
from jax.experimental import pallas as pl

_ALPHA = 4


def _linspace_trunc_idx(t: int) -> tuple:
    with jax.ensure_compile_time_eval():
        v = jnp.linspace(0.0, t - 1, t // _ALPHA).astype(jnp.int32)
    return tuple(int(i) for i in np.asarray(v))


def _gather_body(src_ref, out_ref):
    out_ref[...] = src_ref[...]


def kernel(frames):
    C, T, H, W = frames.shape
    n = T // _ALPHA
    idx = _linspace_trunc_idx(T)
    assert all(i * (T - 1) // (n - 1) == v for i, v in enumerate(idx)), idx

    slow = pl.pallas_call(
        _gather_body,
        grid=(n,),
        in_specs=[
            pl.BlockSpec((C, 1, H, W), lambda t: (0, t * (T - 1) // (n - 1), 0, 0)),
        ],
        out_specs=pl.BlockSpec((C, 1, H, W), lambda t: (0, t, 0, 0)),
        out_shape=jax.ShapeDtypeStruct((C, n, H, W), frames.dtype),
    )(frames)
    return (slow, frames)
